# baseline (speedup 1.0000x reference)
import functools

import jax
import jax.numpy as jnp
from jax.experimental import pallas as pl
from jax.experimental.pallas import tpu as pltpu

_N_ROIS = 20100
_ROWS = 158
_LANES = 128
_PAD = _ROWS * _LANES
_N_GT = 100
_NUM_CLASSES = 21
_K = 256
_FG = 128


def _body(rois_ref, gt_ref, lab_ref, rois_out, labels_out, bbox_out):
    x1 = rois_ref[0]
    y1 = rois_ref[1]
    x2 = rois_ref[2]
    y2 = rois_ref[3]
    area_a = (x2 - x1 + 1.0) * (y2 - y1 + 1.0)
    row_i = jax.lax.broadcasted_iota(jnp.int32, (_ROWS, _LANES), 0)
    col_i = jax.lax.broadcasted_iota(jnp.int32, (_ROWS, _LANES), 1)
    idx = row_i * _LANES + col_i

    def iou_step(j, carry):
        m, amax = carry
        gx1 = gt_ref[j, 0]
        gy1 = gt_ref[j, 1]
        gx2 = gt_ref[j, 2]
        gy2 = gt_ref[j, 3]
        area_b = (gx2 - gx1 + 1.0) * (gy2 - gy1 + 1.0)
        iw = jnp.maximum(jnp.minimum(x2, gx2) - jnp.maximum(x1, gx1) + 1.0, 0.0)
        ih = jnp.maximum(jnp.minimum(y2, gy2) - jnp.maximum(y1, gy1) + 1.0, 0.0)
        inter = iw * ih
        iou = inter / (area_a + area_b - inter)
        upd = iou > m
        return jnp.where(upd, iou, m), jnp.where(upd, j, amax)

    m0 = jnp.full((_ROWS, _LANES), -1.0, jnp.float32)
    a0 = jnp.zeros((_ROWS, _LANES), jnp.int32)
    m, amax = jax.lax.fori_loop(0, _N_GT, iou_step, (m0, a0))
    m = jnp.where(idx < _N_ROIS, m, -1.0)

    cls_iota = jax.lax.broadcasted_iota(jnp.int32, (1, _NUM_CLASSES), 1)
    bb_iota = jax.lax.broadcasted_iota(jnp.int32, (1, 4 * _NUM_CLASSES), 1)

    def sel_step(t, m):
        mval = jnp.max(m)
        cand = m == mval
        sel = jnp.min(jnp.where(cand, idx, jnp.int32(2**30)))
        hot = cand & (idx == sel)
        rx1 = jnp.sum(jnp.where(hot, x1, 0.0))
        ry1 = jnp.sum(jnp.where(hot, y1, 0.0))
        rx2 = jnp.sum(jnp.where(hot, x2, 0.0))
        ry2 = jnp.sum(jnp.where(hot, y2, 0.0))
        a = jnp.sum(jnp.where(hot, amax, 0))
        m = jnp.where(hot, -2.0, m)

        gx1 = gt_ref[a, 0]
        gy1 = gt_ref[a, 1]
        gx2 = gt_ref[a, 2]
        gy2 = gt_ref[a, 3]

        ex_w = rx2 - rx1 + 1.0
        ex_h = ry2 - ry1 + 1.0
        gt_w = gx2 - gx1 + 1.0
        gt_h = gy2 - gy1 + 1.0
        tx = (gx1 + 0.5 * gt_w - rx1 - 0.5 * ex_w) / ex_w
        ty = (gy1 + 0.5 * gt_h - ry1 - 0.5 * ex_h) / ex_h
        tw = jnp.log(gt_w / ex_w)
        th = jnp.log(gt_h / ex_h)

        is_fg = (mval >= 0.5) & (t < _FG)
        lab_row = lab_ref[pl.ds(a, 1), :]
        bg_row = (cls_iota == 0).astype(jnp.float32)
        out_lab = jnp.where(is_fg, lab_row, bg_row)
        cls = jnp.where(is_fg, jnp.sum(jnp.where(lab_row >= 0.5, cls_iota, 0)), 0)

        fgf = is_fg.astype(jnp.float32)
        tsel = bb_iota % 4
        tvals = jnp.where(
            tsel == 0, tx, jnp.where(tsel == 1, ty, jnp.where(tsel == 2, tw, th))
        )
        bb_row = jnp.where(bb_iota // 4 == cls, tvals, 0.0) * fgf

        rois_out[pl.ds(t, 1), :] = jnp.concatenate(
            [rx1.reshape(1, 1), ry1.reshape(1, 1),
             rx2.reshape(1, 1), ry2.reshape(1, 1)], axis=1)
        labels_out[pl.ds(t, 1), :] = out_lab
        bbox_out[pl.ds(t, 1), :] = bb_row
        return m

    jax.lax.fori_loop(0, _K, sel_step, m)


@jax.jit
def kernel(proposals, bounding_boxes, labels):
    rois = jnp.concatenate([proposals[0], bounding_boxes[0]], axis=0)
    pad = jnp.full((_PAD - _N_ROIS, 4), -1e6, jnp.float32)
    planes = jnp.concatenate([rois, pad], axis=0).T.reshape(4, _ROWS, _LANES)
    gt = bounding_boxes[0]
    lab = labels[0]
    out_shape = [
        jax.ShapeDtypeStruct((_K, 4), jnp.float32),
        jax.ShapeDtypeStruct((_K, _NUM_CLASSES), jnp.float32),
        jax.ShapeDtypeStruct((_K, 4 * _NUM_CLASSES), jnp.float32),
    ]
    rois_o, labels_o, bbox_o = pl.pallas_call(
        _body,
        out_shape=out_shape,
        in_specs=[
            pl.BlockSpec(memory_space=pltpu.VMEM),
            pl.BlockSpec(memory_space=pltpu.SMEM),
            pl.BlockSpec(memory_space=pltpu.VMEM),
        ],
        out_specs=[
            pl.BlockSpec(memory_space=pltpu.VMEM),
            pl.BlockSpec(memory_space=pltpu.VMEM),
            pl.BlockSpec(memory_space=pltpu.VMEM),
        ],
    )(planes, gt, lab)
    return rois_o, labels_o, bbox_o


# speedup vs baseline: 1.2261x; 1.2261x over previous
import jax
import jax.numpy as jnp
from jax.experimental import pallas as pl
from jax.experimental.pallas import tpu as pltpu

_N_ROIS = 20100
_ROWS = 158
_LANES = 128
_PAD = _ROWS * _LANES
_N_GT = 100
_NUM_CLASSES = 21
_K = 256
_FG = 128


def _body(rois_ref, gt_ref, gtt_ref, labt_ref, rois_out, labels_out, bbox_out,
          scratch):
    x1 = rois_ref[0]
    y1 = rois_ref[1]
    x2 = rois_ref[2]
    y2 = rois_ref[3]
    area_a = (x2 - x1 + 1.0) * (y2 - y1 + 1.0)
    row_i = jax.lax.broadcasted_iota(jnp.int32, (_ROWS, _LANES), 0)
    col_i = jax.lax.broadcasted_iota(jnp.int32, (_ROWS, _LANES), 1)
    idx = row_i * _LANES + col_i

    def iou_step(j, carry):
        m, amax = carry
        gx1 = gt_ref[j, 0]
        gy1 = gt_ref[j, 1]
        gx2 = gt_ref[j, 2]
        gy2 = gt_ref[j, 3]
        area_b = (gx2 - gx1 + 1.0) * (gy2 - gy1 + 1.0)
        iw = jnp.maximum(jnp.minimum(x2, gx2) - jnp.maximum(x1, gx1) + 1.0, 0.0)
        ih = jnp.maximum(jnp.minimum(y2, gy2) - jnp.maximum(y1, gy1) + 1.0, 0.0)
        inter = iw * ih
        iou = inter / (area_a + area_b - inter)
        upd = iou > m
        return jnp.where(upd, iou, m), jnp.where(upd, j.astype(jnp.float32), amax)

    m0 = jnp.full((_ROWS, _LANES), -1.0, jnp.float32)
    a0 = jnp.zeros((_ROWS, _LANES), jnp.float32)
    m, amaxf = jax.lax.fori_loop(0, _N_GT, iou_step, (m0, a0))
    m = jnp.where(idx < _N_ROIS, m, -1.0)

    def sel_step(t, m):
        mval = jnp.max(m, axis=(0, 1), keepdims=True)
        cand = m == mval
        midx = jnp.min(jnp.where(cand, idx, jnp.int32(2**30)),
                       axis=(0, 1), keepdims=True)
        hot = cand & (idx == midx)

        def ex(v):
            return jnp.sum(jnp.where(hot, v, 0.0), axis=(0, 1), keepdims=True)

        rec = jnp.concatenate(
            [ex(x1), ex(y1), ex(x2), ex(y2), ex(amaxf), mval], axis=1)
        scratch[pl.ds(t, 1), :] = rec
        return jnp.where(hot, -2.0, m)

    jax.lax.fori_loop(0, _K, sel_step, m)

    sx1 = scratch[:, 0:1]
    sy1 = scratch[:, 1:2]
    sx2 = scratch[:, 2:3]
    sy2 = scratch[:, 3:4]
    a_f = scratch[:, 4:5]
    mv = scratch[:, 5:6]

    gt_iota = jax.lax.broadcasted_iota(jnp.int32, (1, _N_GT), 1).astype(jnp.float32)
    onehot = a_f == gt_iota

    def gsum(rowvec):
        return jnp.sum(jnp.where(onehot, rowvec, 0.0), axis=1, keepdims=True)

    gx1 = gsum(gtt_ref[0:1, :])
    gy1 = gsum(gtt_ref[1:2, :])
    gx2 = gsum(gtt_ref[2:3, :])
    gy2 = gsum(gtt_ref[3:4, :])
    cls_col = jax.lax.broadcasted_iota(
        jnp.int32, (_NUM_CLASSES, 1), 0).astype(jnp.float32)
    clsvec = jnp.sum(labt_ref[:, :] * cls_col, axis=0, keepdims=True)
    gcls = gsum(clsvec)

    ex_w = sx2 - sx1 + 1.0
    ex_h = sy2 - sy1 + 1.0
    gt_w = gx2 - gx1 + 1.0
    gt_h = gy2 - gy1 + 1.0
    tx = (gx1 + 0.5 * gt_w - sx1 - 0.5 * ex_w) / ex_w
    ty = (gy1 + 0.5 * gt_h - sy1 - 0.5 * ex_h) / ex_h
    tw = jnp.log(gt_w / ex_w)
    th = jnp.log(gt_h / ex_h)

    rank = jax.lax.broadcasted_iota(jnp.int32, (_K, 1), 0)
    is_fg = (mv >= 0.5) & (rank < _FG)

    cls_iota = jax.lax.broadcasted_iota(
        jnp.int32, (1, _NUM_CLASSES), 1).astype(jnp.float32)
    glab = (gcls == cls_iota).astype(jnp.float32)
    bg_row = (cls_iota == 0.0).astype(jnp.float32)
    labels_out[:, :] = jnp.where(is_fg, glab, bg_row)

    cls_i = jnp.where(is_fg, gcls.astype(jnp.int32), 0)

    bb_iota = jax.lax.broadcasted_iota(jnp.int32, (1, 4 * _NUM_CLASSES), 1)
    tsel = bb_iota % 4
    tvals = jnp.where(tsel == 0, tx,
                      jnp.where(tsel == 1, ty,
                                jnp.where(tsel == 2, tw, th)))
    bbox_out[:, :] = jnp.where((bb_iota // 4 == cls_i) & is_fg, tvals, 0.0)

    rois_out[:, :] = jnp.concatenate([sx1, sy1, sx2, sy2], axis=1)


@jax.jit
def kernel(proposals, bounding_boxes, labels):
    rois = jnp.concatenate([proposals[0], bounding_boxes[0]], axis=0)
    pad = jnp.full((_PAD - _N_ROIS, 4), -1e6, jnp.float32)
    planes = jnp.concatenate([rois, pad], axis=0).T.reshape(4, _ROWS, _LANES)
    gt = bounding_boxes[0]
    gt_t = gt.T
    lab_t = labels[0].T
    out_shape = [
        jax.ShapeDtypeStruct((_K, 4), jnp.float32),
        jax.ShapeDtypeStruct((_K, _NUM_CLASSES), jnp.float32),
        jax.ShapeDtypeStruct((_K, 4 * _NUM_CLASSES), jnp.float32),
    ]
    rois_o, labels_o, bbox_o = pl.pallas_call(
        _body,
        out_shape=out_shape,
        in_specs=[
            pl.BlockSpec(memory_space=pltpu.VMEM),
            pl.BlockSpec(memory_space=pltpu.SMEM),
            pl.BlockSpec(memory_space=pltpu.VMEM),
            pl.BlockSpec(memory_space=pltpu.VMEM),
        ],
        out_specs=[
            pl.BlockSpec(memory_space=pltpu.VMEM),
            pl.BlockSpec(memory_space=pltpu.VMEM),
            pl.BlockSpec(memory_space=pltpu.VMEM),
        ],
        scratch_shapes=[pltpu.VMEM((_K, 6), jnp.float32)],
    )(planes, gt, gt_t, lab_t)
    return rois_o, labels_o, bbox_o


# speedup vs baseline: 1.7642x; 1.4388x over previous
import jax
import jax.numpy as jnp
from jax.experimental import pallas as pl
from jax.experimental.pallas import tpu as pltpu

_N_ROIS = 20100
_ROWS = 256
_LANES = 128
_PAD = _ROWS * _LANES
_N_GT = 100
_NUM_CLASSES = 21
_K = 256
_FG = 128


def _rank_before(am, ap, bm, bp):
    return (am > bm) | ((am == bm) & (ap < bp))


def _cmpx(arrs, j, dirm_row=None, dirm_col=None):
    r, w = arrs[0].shape
    g = r // (2 * j)
    quads = [x.reshape(g, 2, j, w) for x in arrs]
    a = [q[:, 0] for q in quads]
    b = [q[:, 1] for q in quads]
    rb = _rank_before(a[0], a[1], b[0], b[1])
    if dirm_row is not None:
        sel = rb == dirm_row.reshape(g, 2, j, 1)[:, 0]
    elif dirm_col is not None:
        sel = rb == dirm_col
    else:
        sel = rb
    out = []
    for x, y in zip(a, b):
        na = jnp.where(sel, x, y)
        nb = jnp.where(sel, y, x)
        out.append(jnp.stack((na, nb), axis=1).reshape(r, w))
    return out


def _body(rois_ref, gt_ref, gtt_ref, labt_ref, rois_out, labels_out, bbox_out):
    x1 = rois_ref[0]
    y1 = rois_ref[1]
    x2 = rois_ref[2]
    y2 = rois_ref[3]
    area_a = (x2 - x1 + 1.0) * (y2 - y1 + 1.0)
    row_i = jax.lax.broadcasted_iota(jnp.int32, (_ROWS, _LANES), 0)
    col_i = jax.lax.broadcasted_iota(jnp.int32, (_ROWS, _LANES), 1)
    idx = row_i * _LANES + col_i

    def iou_step(j, carry):
        m, amax = carry
        gx1 = gt_ref[j, 0]
        gy1 = gt_ref[j, 1]
        gx2 = gt_ref[j, 2]
        gy2 = gt_ref[j, 3]
        area_b = (gx2 - gx1 + 1.0) * (gy2 - gy1 + 1.0)
        iw = jnp.maximum(jnp.minimum(x2, gx2) - jnp.maximum(x1, gx1) + 1.0, 0.0)
        ih = jnp.maximum(jnp.minimum(y2, gy2) - jnp.maximum(y1, gy1) + 1.0, 0.0)
        inter = iw * ih
        iou = inter / (area_a + area_b - inter)
        upd = iou > m
        return jnp.where(upd, iou, m), jnp.where(upd, j, amax)

    m0 = jnp.full((_ROWS, _LANES), -1.0, jnp.float32)
    a0 = jnp.zeros((_ROWS, _LANES), jnp.int32)
    m, amax = jax.lax.fori_loop(0, _N_GT, iou_step, (m0, a0))
    m = jnp.where(idx < _N_ROIS, m, -1.0)
    packed = idx * _LANES + amax

    arrs = [m, packed, x1, y1, x2, y2]
    riota = jax.lax.broadcasted_iota(jnp.int32, (_ROWS, 1), 0)
    k = 2
    while k <= _ROWS:
        if k < _ROWS:
            kw = dict(dirm_row=(riota & k) == 0)
        else:
            col = jax.lax.broadcasted_iota(jnp.int32, (1, _LANES), 1)
            kw = dict(dirm_col=col < _LANES // 2)
        j = k // 2
        while j >= 1:
            arrs = _cmpx(arrs, j, **kw)
            j //= 2
        k *= 2

    w = _LANES
    while w > 1:
        w2 = w // 2
        a = [x[:, :w2] for x in arrs]
        b = [x[:, w2:] for x in arrs]
        rb = _rank_before(a[0], a[1], b[0], b[1])
        arrs = [jnp.where(rb, x, y) for x, y in zip(a, b)]
        colw = jax.lax.broadcasted_iota(jnp.int32, (1, w2), 1)
        dirm_col = colw < max(1, w2 // 2)
        j = _ROWS // 2
        while j >= 1:
            arrs = _cmpx(arrs, j, dirm_col=dirm_col)
            j //= 2
        w = w2

    m_s, packed_s, sx1, sy1, sx2, sy2 = arrs

    a_f = (packed_s % _LANES).astype(jnp.float32)

    gt_iota = jax.lax.broadcasted_iota(jnp.int32, (1, _N_GT), 1).astype(jnp.float32)
    onehot = a_f == gt_iota

    def gsum(rowvec):
        return jnp.sum(jnp.where(onehot, rowvec, 0.0), axis=1, keepdims=True)

    gx1 = gsum(gtt_ref[0:1, :])
    gy1 = gsum(gtt_ref[1:2, :])
    gx2 = gsum(gtt_ref[2:3, :])
    gy2 = gsum(gtt_ref[3:4, :])
    cls_col = jax.lax.broadcasted_iota(
        jnp.int32, (_NUM_CLASSES, 1), 0).astype(jnp.float32)
    clsvec = jnp.sum(labt_ref[:, :] * cls_col, axis=0, keepdims=True)
    gcls = gsum(clsvec)

    ex_w = sx2 - sx1 + 1.0
    ex_h = sy2 - sy1 + 1.0
    gt_w = gx2 - gx1 + 1.0
    gt_h = gy2 - gy1 + 1.0
    tx = (gx1 + 0.5 * gt_w - sx1 - 0.5 * ex_w) / ex_w
    ty = (gy1 + 0.5 * gt_h - sy1 - 0.5 * ex_h) / ex_h
    tw = jnp.log(gt_w / ex_w)
    th = jnp.log(gt_h / ex_h)

    rank = jax.lax.broadcasted_iota(jnp.int32, (_K, 1), 0)
    is_fg = (m_s >= 0.5) & (rank < _FG)

    cls_iota = jax.lax.broadcasted_iota(
        jnp.int32, (1, _NUM_CLASSES), 1).astype(jnp.float32)
    glab = (gcls == cls_iota).astype(jnp.float32)
    bg_row = (cls_iota == 0.0).astype(jnp.float32)
    labels_out[:, :] = jnp.where(is_fg, glab, bg_row)

    cls_i = jnp.where(is_fg, gcls.astype(jnp.int32), 0)

    bb_iota = jax.lax.broadcasted_iota(jnp.int32, (1, 4 * _NUM_CLASSES), 1)
    tsel = bb_iota % 4
    tvals = jnp.where(tsel == 0, tx,
                      jnp.where(tsel == 1, ty,
                                jnp.where(tsel == 2, tw, th)))
    bbox_out[:, :] = jnp.where((bb_iota // 4 == cls_i) & is_fg, tvals, 0.0)

    rois_out[:, :] = jnp.concatenate([sx1, sy1, sx2, sy2], axis=1)


@jax.jit
def kernel(proposals, bounding_boxes, labels):
    rois = jnp.concatenate([proposals[0], bounding_boxes[0]], axis=0)
    pad = jnp.full((_PAD - _N_ROIS, 4), -1e6, jnp.float32)
    planes = jnp.concatenate([rois, pad], axis=0).T.reshape(4, _ROWS, _LANES)
    gt = bounding_boxes[0]
    gt_t = gt.T
    lab_t = labels[0].T
    out_shape = [
        jax.ShapeDtypeStruct((_K, 4), jnp.float32),
        jax.ShapeDtypeStruct((_K, _NUM_CLASSES), jnp.float32),
        jax.ShapeDtypeStruct((_K, 4 * _NUM_CLASSES), jnp.float32),
    ]
    rois_o, labels_o, bbox_o = pl.pallas_call(
        _body,
        out_shape=out_shape,
        in_specs=[
            pl.BlockSpec(memory_space=pltpu.VMEM),
            pl.BlockSpec(memory_space=pltpu.SMEM),
            pl.BlockSpec(memory_space=pltpu.VMEM),
            pl.BlockSpec(memory_space=pltpu.VMEM),
        ],
        out_specs=[
            pl.BlockSpec(memory_space=pltpu.VMEM),
            pl.BlockSpec(memory_space=pltpu.VMEM),
            pl.BlockSpec(memory_space=pltpu.VMEM),
        ],
    )(planes, gt, gt_t, lab_t)
    return rois_o, labels_o, bbox_o


# speedup vs baseline: 4.8898x; 2.7716x over previous
import jax
import jax.numpy as jnp
from jax.experimental import pallas as pl
from jax.experimental.pallas import tpu as pltpu

_N_ROIS = 20100
_ROWS = 256
_LANES = 128
_PAD = _ROWS * _LANES
_N_GT = 100
_NUM_CLASSES = 21
_K = 256
_FG = 128


def _rank_before(am, ap, bm, bp):
    return (am > bm) | ((am == bm) & (ap < bp))


def _cmpx(arrs, j, riota, dirm):
    r = arrs[0].shape[0]
    first = (riota & j) == 0
    parts = [jnp.where(first, pltpu.roll(x, r - j, 0), pltpu.roll(x, j, 0))
             for x in arrs]
    rb = _rank_before(arrs[0], arrs[1], parts[0], parts[1])
    keep = (rb == first) if dirm is None else (rb == (first == dirm))
    return [jnp.where(keep, x, p) for x, p in zip(arrs, parts)]


def _body(rois_ref, gt_ref, gtt_ref, labt_ref, rois_out, labels_out, bbox_out):
    x1 = rois_ref[0]
    y1 = rois_ref[1]
    x2 = rois_ref[2]
    y2 = rois_ref[3]
    area_a = (x2 - x1 + 1.0) * (y2 - y1 + 1.0)
    row_i = jax.lax.broadcasted_iota(jnp.int32, (_ROWS, _LANES), 0)
    col_i = jax.lax.broadcasted_iota(jnp.int32, (_ROWS, _LANES), 1)
    idx = row_i * _LANES + col_i

    def iou_step(j, carry):
        m, amax = carry
        gx1 = gt_ref[j, 0]
        gy1 = gt_ref[j, 1]
        gx2 = gt_ref[j, 2]
        gy2 = gt_ref[j, 3]
        area_b = (gx2 - gx1 + 1.0) * (gy2 - gy1 + 1.0)
        iw = jnp.maximum(jnp.minimum(x2, gx2) - jnp.maximum(x1, gx1) + 1.0, 0.0)
        ih = jnp.maximum(jnp.minimum(y2, gy2) - jnp.maximum(y1, gy1) + 1.0, 0.0)
        inter = iw * ih
        iou = inter / (area_a + area_b - inter)
        upd = iou > m
        return jnp.where(upd, iou, m), jnp.where(upd, j, amax)

    m0 = jnp.full((_ROWS, _LANES), -1.0, jnp.float32)
    a0 = jnp.zeros((_ROWS, _LANES), jnp.int32)
    m, amax = jax.lax.fori_loop(0, _N_GT, iou_step, (m0, a0))
    m = jnp.where(idx < _N_ROIS, m, -1.0)
    packed = idx * _LANES + amax

    arrs = [m, packed]
    riota = jax.lax.broadcasted_iota(jnp.int32, (_ROWS, 1), 0)
    k = 2
    while k <= _ROWS:
        if k < _ROWS:
            dirm = (riota & k) == 0
        else:
            col = jax.lax.broadcasted_iota(jnp.int32, (1, _LANES), 1)
            dirm = col < _LANES // 2
        j = k // 2
        while j >= 1:
            arrs = _cmpx(arrs, j, riota, dirm)
            j //= 2
        k *= 2

    w = _LANES
    while w > 1:
        w2 = w // 2
        a = [x[:, :w2] for x in arrs]
        b = [x[:, w2:] for x in arrs]
        rb = _rank_before(a[0], a[1], b[0], b[1])
        arrs = [jnp.where(rb, x, y) for x, y in zip(a, b)]
        colw = jax.lax.broadcasted_iota(jnp.int32, (1, w2), 1)
        dirm_col = colw < max(1, w2 // 2)
        j = _ROWS // 2
        while j >= 1:
            arrs = _cmpx(arrs, j, riota, dirm_col)
            j //= 2
        w = w2

    m_s, packed_s = arrs

    t_idx = packed_s // _LANES
    tr = t_idx // _LANES
    tc = t_idx % _LANES
    rowoh = (tr == jax.lax.broadcasted_iota(jnp.int32, (1, _ROWS), 1)
             ).astype(jnp.bfloat16)
    colmask = tc == jax.lax.broadcasted_iota(jnp.int32, (1, _LANES), 1)

    def gplane(xp):
        hi = xp.astype(jnp.bfloat16)
        r1 = xp - hi.astype(jnp.float32)
        mid = r1.astype(jnp.bfloat16)
        lo = (r1 - mid.astype(jnp.float32)).astype(jnp.bfloat16)
        gh = jnp.dot(rowoh, hi, preferred_element_type=jnp.float32)
        gm = jnp.dot(rowoh, mid, preferred_element_type=jnp.float32)
        gl = jnp.dot(rowoh, lo, preferred_element_type=jnp.float32)
        rows = gh + (gm + gl)
        return jnp.sum(jnp.where(colmask, rows, 0.0), axis=1, keepdims=True)

    sx1 = gplane(x1)
    sy1 = gplane(y1)
    sx2 = gplane(x2)
    sy2 = gplane(y2)

    a_f = (packed_s % _LANES).astype(jnp.float32)

    gt_iota = jax.lax.broadcasted_iota(jnp.int32, (1, _N_GT), 1).astype(jnp.float32)
    onehot = a_f == gt_iota

    def gsum(rowvec):
        return jnp.sum(jnp.where(onehot, rowvec, 0.0), axis=1, keepdims=True)

    gx1 = gsum(gtt_ref[0:1, :])
    gy1 = gsum(gtt_ref[1:2, :])
    gx2 = gsum(gtt_ref[2:3, :])
    gy2 = gsum(gtt_ref[3:4, :])
    cls_col = jax.lax.broadcasted_iota(
        jnp.int32, (_NUM_CLASSES, 1), 0).astype(jnp.float32)
    clsvec = jnp.sum(labt_ref[:, :] * cls_col, axis=0, keepdims=True)
    gcls = gsum(clsvec)

    ex_w = sx2 - sx1 + 1.0
    ex_h = sy2 - sy1 + 1.0
    gt_w = gx2 - gx1 + 1.0
    gt_h = gy2 - gy1 + 1.0
    tx = (gx1 + 0.5 * gt_w - sx1 - 0.5 * ex_w) / ex_w
    ty = (gy1 + 0.5 * gt_h - sy1 - 0.5 * ex_h) / ex_h
    tw = jnp.log(gt_w / ex_w)
    th = jnp.log(gt_h / ex_h)

    rank = jax.lax.broadcasted_iota(jnp.int32, (_K, 1), 0)
    is_fg = (m_s >= 0.5) & (rank < _FG)

    cls_iota = jax.lax.broadcasted_iota(
        jnp.int32, (1, _NUM_CLASSES), 1).astype(jnp.float32)
    glab = (gcls == cls_iota).astype(jnp.float32)
    bg_row = (cls_iota == 0.0).astype(jnp.float32)
    labels_out[:, :] = jnp.where(is_fg, glab, bg_row)

    cls_i = jnp.where(is_fg, gcls.astype(jnp.int32), 0)

    bb_iota = jax.lax.broadcasted_iota(jnp.int32, (1, 4 * _NUM_CLASSES), 1)
    tsel = bb_iota % 4
    tvals = jnp.where(tsel == 0, tx,
                      jnp.where(tsel == 1, ty,
                                jnp.where(tsel == 2, tw, th)))
    bbox_out[:, :] = jnp.where((bb_iota // 4 == cls_i) & is_fg, tvals, 0.0)

    rois_out[:, :] = jnp.concatenate([sx1, sy1, sx2, sy2], axis=1)


@jax.jit
def kernel(proposals, bounding_boxes, labels):
    rois = jnp.concatenate([proposals[0], bounding_boxes[0]], axis=0)
    pad = jnp.full((_PAD - _N_ROIS, 4), -1e6, jnp.float32)
    planes = jnp.concatenate([rois, pad], axis=0).T.reshape(4, _ROWS, _LANES)
    gt = bounding_boxes[0]
    gt_t = gt.T
    lab_t = labels[0].T
    out_shape = [
        jax.ShapeDtypeStruct((_K, 4), jnp.float32),
        jax.ShapeDtypeStruct((_K, _NUM_CLASSES), jnp.float32),
        jax.ShapeDtypeStruct((_K, 4 * _NUM_CLASSES), jnp.float32),
    ]
    rois_o, labels_o, bbox_o = pl.pallas_call(
        _body,
        out_shape=out_shape,
        in_specs=[
            pl.BlockSpec(memory_space=pltpu.VMEM),
            pl.BlockSpec(memory_space=pltpu.SMEM),
            pl.BlockSpec(memory_space=pltpu.VMEM),
            pl.BlockSpec(memory_space=pltpu.VMEM),
        ],
        out_specs=[
            pl.BlockSpec(memory_space=pltpu.VMEM),
            pl.BlockSpec(memory_space=pltpu.VMEM),
            pl.BlockSpec(memory_space=pltpu.VMEM),
        ],
    )(planes, gt, gt_t, lab_t)
    return rois_o, labels_o, bbox_o


# speedup vs baseline: 5.0573x; 1.0343x over previous
import jax
import jax.numpy as jnp
from jax.experimental import pallas as pl
from jax.experimental.pallas import tpu as pltpu

_N_ROIS = 20100
_ROWS = 256
_LANES = 128
_PAD = _ROWS * _LANES
_N_GT = 100
_NUM_CLASSES = 21
_K = 256
_FG = 128


def _rank_before(am, ap, bm, bp):
    return (am > bm) | ((am == bm) & (ap < bp))


def _cmpx(arrs, j, riota, dirm):
    r = arrs[0].shape[0]
    first = (riota & j) == 0
    parts = [jnp.where(first, pltpu.roll(x, r - j, 0), pltpu.roll(x, j, 0))
             for x in arrs]
    rb = _rank_before(arrs[0], arrs[1], parts[0], parts[1])
    keep = (rb == first) if dirm is None else (rb == (first == dirm))
    return [jnp.where(keep, x, p) for x, p in zip(arrs, parts)]


def _body(rois_ref, gt_ref, gtt_ref, labt_ref, rois_out, labels_out, bbox_out):
    x1 = rois_ref[0]
    y1 = rois_ref[1]
    x2 = rois_ref[2]
    y2 = rois_ref[3]
    area_a = (x2 - x1 + 1.0) * (y2 - y1 + 1.0)
    row_i = jax.lax.broadcasted_iota(jnp.int32, (_ROWS, _LANES), 0)
    col_i = jax.lax.broadcasted_iota(jnp.int32, (_ROWS, _LANES), 1)
    idx = row_i * _LANES + col_i

    def iou_step(j, carry):
        m, amax = carry
        gx1 = gt_ref[j, 0]
        gy1 = gt_ref[j, 1]
        gx2 = gt_ref[j, 2]
        gy2 = gt_ref[j, 3]
        area_b = (gx2 - gx1 + 1.0) * (gy2 - gy1 + 1.0)
        iw = jnp.maximum(jnp.minimum(x2, gx2) - jnp.maximum(x1, gx1) + 1.0, 0.0)
        ih = jnp.maximum(jnp.minimum(y2, gy2) - jnp.maximum(y1, gy1) + 1.0, 0.0)
        inter = iw * ih
        iou = inter / (area_a + area_b - inter)
        upd = iou > m
        return jnp.where(upd, iou, m), jnp.where(upd, j, amax)

    m0 = jnp.full((_ROWS, _LANES), -1.0, jnp.float32)
    a0 = jnp.zeros((_ROWS, _LANES), jnp.int32)
    m, amax = jax.lax.fori_loop(0, _N_GT, iou_step, (m0, a0), unroll=10)
    m = jnp.where(idx < _N_ROIS, m, -1.0)
    packed = idx * _LANES + amax

    arrs = [m, packed]
    riota = jax.lax.broadcasted_iota(jnp.int32, (_ROWS, 1), 0)
    k = 2
    while k <= _ROWS:
        if k < _ROWS:
            dirm = (riota & k) == 0
        else:
            col = jax.lax.broadcasted_iota(jnp.int32, (1, _LANES), 1)
            dirm = col < _LANES // 2
        j = k // 2
        while j >= 1:
            arrs = _cmpx(arrs, j, riota, dirm)
            j //= 2
        k *= 2

    w = _LANES
    while w > 1:
        w2 = w // 2
        a = [x[:, :w2] for x in arrs]
        b = [x[:, w2:] for x in arrs]
        rb = _rank_before(a[0], a[1], b[0], b[1])
        arrs = [jnp.where(rb, x, y) for x, y in zip(a, b)]
        colw = jax.lax.broadcasted_iota(jnp.int32, (1, w2), 1)
        dirm_col = colw < max(1, w2 // 2)
        j = _ROWS // 2
        while j >= 1:
            arrs = _cmpx(arrs, j, riota, dirm_col)
            j //= 2
        w = w2

    m_s, packed_s = arrs

    t_idx = packed_s // _LANES
    tr = t_idx // _LANES
    tc = t_idx % _LANES
    rowoh = (tr == jax.lax.broadcasted_iota(jnp.int32, (1, _ROWS), 1)
             ).astype(jnp.bfloat16)
    colmask = tc == jax.lax.broadcasted_iota(jnp.int32, (1, _LANES), 1)

    def gplane(xp):
        hi = xp.astype(jnp.bfloat16)
        r1 = xp - hi.astype(jnp.float32)
        mid = r1.astype(jnp.bfloat16)
        lo = (r1 - mid.astype(jnp.float32)).astype(jnp.bfloat16)
        gh = jnp.dot(rowoh, hi, preferred_element_type=jnp.float32)
        gm = jnp.dot(rowoh, mid, preferred_element_type=jnp.float32)
        gl = jnp.dot(rowoh, lo, preferred_element_type=jnp.float32)
        rows = gh + (gm + gl)
        return jnp.sum(jnp.where(colmask, rows, 0.0), axis=1, keepdims=True)

    sx1 = gplane(x1)
    sy1 = gplane(y1)
    sx2 = gplane(x2)
    sy2 = gplane(y2)

    a_f = (packed_s % _LANES).astype(jnp.float32)

    gt_iota = jax.lax.broadcasted_iota(jnp.int32, (1, _N_GT), 1).astype(jnp.float32)
    onehot = a_f == gt_iota

    def gsum(rowvec):
        return jnp.sum(jnp.where(onehot, rowvec, 0.0), axis=1, keepdims=True)

    gx1 = gsum(gtt_ref[0:1, :])
    gy1 = gsum(gtt_ref[1:2, :])
    gx2 = gsum(gtt_ref[2:3, :])
    gy2 = gsum(gtt_ref[3:4, :])
    cls_col = jax.lax.broadcasted_iota(
        jnp.int32, (_NUM_CLASSES, 1), 0).astype(jnp.float32)
    clsvec = jnp.sum(labt_ref[:, :] * cls_col, axis=0, keepdims=True)
    gcls = gsum(clsvec)

    ex_w = sx2 - sx1 + 1.0
    ex_h = sy2 - sy1 + 1.0
    gt_w = gx2 - gx1 + 1.0
    gt_h = gy2 - gy1 + 1.0
    tx = (gx1 + 0.5 * gt_w - sx1 - 0.5 * ex_w) / ex_w
    ty = (gy1 + 0.5 * gt_h - sy1 - 0.5 * ex_h) / ex_h
    tw = jnp.log(gt_w / ex_w)
    th = jnp.log(gt_h / ex_h)

    rank = jax.lax.broadcasted_iota(jnp.int32, (_K, 1), 0)
    is_fg = (m_s >= 0.5) & (rank < _FG)

    cls_iota = jax.lax.broadcasted_iota(
        jnp.int32, (1, _NUM_CLASSES), 1).astype(jnp.float32)
    glab = (gcls == cls_iota).astype(jnp.float32)
    bg_row = (cls_iota == 0.0).astype(jnp.float32)
    labels_out[:, :] = jnp.where(is_fg, glab, bg_row)

    cls_i = jnp.where(is_fg, gcls.astype(jnp.int32), 0)

    bb_iota = jax.lax.broadcasted_iota(jnp.int32, (1, 4 * _NUM_CLASSES), 1)
    tsel = bb_iota % 4
    tvals = jnp.where(tsel == 0, tx,
                      jnp.where(tsel == 1, ty,
                                jnp.where(tsel == 2, tw, th)))
    bbox_out[:, :] = jnp.where((bb_iota // 4 == cls_i) & is_fg, tvals, 0.0)

    rois_out[:, :] = jnp.concatenate([sx1, sy1, sx2, sy2], axis=1)


@jax.jit
def kernel(proposals, bounding_boxes, labels):
    rois = jnp.concatenate([proposals[0], bounding_boxes[0]], axis=0)
    pad = jnp.full((_PAD - _N_ROIS, 4), -1e6, jnp.float32)
    planes = jnp.concatenate([rois, pad], axis=0).T.reshape(4, _ROWS, _LANES)
    gt = bounding_boxes[0]
    gt_t = gt.T
    lab_t = labels[0].T
    out_shape = [
        jax.ShapeDtypeStruct((_K, 4), jnp.float32),
        jax.ShapeDtypeStruct((_K, _NUM_CLASSES), jnp.float32),
        jax.ShapeDtypeStruct((_K, 4 * _NUM_CLASSES), jnp.float32),
    ]
    rois_o, labels_o, bbox_o = pl.pallas_call(
        _body,
        out_shape=out_shape,
        in_specs=[
            pl.BlockSpec(memory_space=pltpu.VMEM),
            pl.BlockSpec(memory_space=pltpu.SMEM),
            pl.BlockSpec(memory_space=pltpu.VMEM),
            pl.BlockSpec(memory_space=pltpu.VMEM),
        ],
        out_specs=[
            pl.BlockSpec(memory_space=pltpu.VMEM),
            pl.BlockSpec(memory_space=pltpu.VMEM),
            pl.BlockSpec(memory_space=pltpu.VMEM),
        ],
    )(planes, gt, gt_t, lab_t)
    return rois_o, labels_o, bbox_o


# speedup vs baseline: 5.0764x; 1.0038x over previous
import jax
import jax.numpy as jnp
from jax.experimental import pallas as pl
from jax.experimental.pallas import tpu as pltpu

_N_ROIS = 20100
_ROWS = 256
_LANES = 128
_PAD = _ROWS * _LANES
_N_GT = 100
_NUM_CLASSES = 21
_K = 256
_FG = 128


def _rank_before(am, ap, bm, bp):
    return (am > bm) | ((am == bm) & (ap < bp))


def _cmpx(arrs, j, riota, dirm):
    r = arrs[0].shape[0]
    first = (riota & j) == 0
    parts = [jnp.where(first, pltpu.roll(x, r - j, 0), pltpu.roll(x, j, 0))
             for x in arrs]
    rb = _rank_before(arrs[0], arrs[1], parts[0], parts[1])
    keep = (rb == first) if dirm is None else (rb == (first == dirm))
    return [jnp.where(keep, x, p) for x, p in zip(arrs, parts)]


def _body(rois_ref, gt_ref, gtt_ref, labt_ref, rois_out, labels_out, bbox_out):
    x1 = rois_ref[0]
    y1 = rois_ref[1]
    x2 = rois_ref[2]
    y2 = rois_ref[3]
    area_a = (x2 - x1 + 1.0) * (y2 - y1 + 1.0)
    row_i = jax.lax.broadcasted_iota(jnp.int32, (_ROWS, _LANES), 0)
    col_i = jax.lax.broadcasted_iota(jnp.int32, (_ROWS, _LANES), 1)
    idx = row_i * _LANES + col_i

    _CH = 32
    m_parts, a_parts = [], []
    for c in range(0, _ROWS, _CH):
        cx1 = x1[c:c + _CH]
        cy1 = y1[c:c + _CH]
        cx2 = x2[c:c + _CH]
        cy2 = y2[c:c + _CH]
        c_area = area_a[c:c + _CH]

        def iou_step(j, carry, cx1=cx1, cy1=cy1, cx2=cx2, cy2=cy2,
                     c_area=c_area):
            m, amax = carry
            gx1 = gt_ref[j, 0]
            gy1 = gt_ref[j, 1]
            gx2 = gt_ref[j, 2]
            gy2 = gt_ref[j, 3]
            area_b = (gx2 - gx1 + 1.0) * (gy2 - gy1 + 1.0)
            iw = jnp.maximum(
                jnp.minimum(cx2, gx2) - jnp.maximum(cx1, gx1) + 1.0, 0.0)
            ih = jnp.maximum(
                jnp.minimum(cy2, gy2) - jnp.maximum(cy1, gy1) + 1.0, 0.0)
            inter = iw * ih
            iou = inter / (c_area + area_b - inter)
            upd = iou > m
            return jnp.where(upd, iou, m), jnp.where(upd, j, amax)

        m0 = jnp.full((_CH, _LANES), -1.0, jnp.float32)
        a0 = jnp.zeros((_CH, _LANES), jnp.int32)
        mc, ac = jax.lax.fori_loop(0, _N_GT, iou_step, (m0, a0), unroll=4)
        m_parts.append(mc)
        a_parts.append(ac)
    m = jnp.concatenate(m_parts, axis=0)
    amax = jnp.concatenate(a_parts, axis=0)
    m = jnp.where(idx < _N_ROIS, m, -1.0)
    packed = idx * _LANES + amax

    arrs = [m, packed]
    riota = jax.lax.broadcasted_iota(jnp.int32, (_ROWS, 1), 0)
    k = 2
    while k <= _ROWS:
        if k < _ROWS:
            dirm = (riota & k) == 0
        else:
            col = jax.lax.broadcasted_iota(jnp.int32, (1, _LANES), 1)
            dirm = col < _LANES // 2
        j = k // 2
        while j >= 1:
            arrs = _cmpx(arrs, j, riota, dirm)
            j //= 2
        k *= 2

    w = _LANES
    while w > 1:
        w2 = w // 2
        a = [x[:, :w2] for x in arrs]
        b = [x[:, w2:] for x in arrs]
        rb = _rank_before(a[0], a[1], b[0], b[1])
        arrs = [jnp.where(rb, x, y) for x, y in zip(a, b)]
        colw = jax.lax.broadcasted_iota(jnp.int32, (1, w2), 1)
        dirm_col = colw < max(1, w2 // 2)
        j = _ROWS // 2
        while j >= 1:
            arrs = _cmpx(arrs, j, riota, dirm_col)
            j //= 2
        w = w2

    m_s, packed_s = arrs

    t_idx = packed_s // _LANES
    tr = t_idx // _LANES
    tc = t_idx % _LANES
    rowoh = (tr == jax.lax.broadcasted_iota(jnp.int32, (1, _ROWS), 1)
             ).astype(jnp.bfloat16)
    colmask = tc == jax.lax.broadcasted_iota(jnp.int32, (1, _LANES), 1)

    def gplane(xp):
        hi = xp.astype(jnp.bfloat16)
        r1 = xp - hi.astype(jnp.float32)
        mid = r1.astype(jnp.bfloat16)
        lo = (r1 - mid.astype(jnp.float32)).astype(jnp.bfloat16)
        gh = jnp.dot(rowoh, hi, preferred_element_type=jnp.float32)
        gm = jnp.dot(rowoh, mid, preferred_element_type=jnp.float32)
        gl = jnp.dot(rowoh, lo, preferred_element_type=jnp.float32)
        rows = gh + (gm + gl)
        return jnp.sum(jnp.where(colmask, rows, 0.0), axis=1, keepdims=True)

    sx1 = gplane(x1)
    sy1 = gplane(y1)
    sx2 = gplane(x2)
    sy2 = gplane(y2)

    a_f = (packed_s % _LANES).astype(jnp.float32)

    gt_iota = jax.lax.broadcasted_iota(jnp.int32, (1, _N_GT), 1).astype(jnp.float32)
    onehot = a_f == gt_iota

    def gsum(rowvec):
        return jnp.sum(jnp.where(onehot, rowvec, 0.0), axis=1, keepdims=True)

    gx1 = gsum(gtt_ref[0:1, :])
    gy1 = gsum(gtt_ref[1:2, :])
    gx2 = gsum(gtt_ref[2:3, :])
    gy2 = gsum(gtt_ref[3:4, :])
    cls_col = jax.lax.broadcasted_iota(
        jnp.int32, (_NUM_CLASSES, 1), 0).astype(jnp.float32)
    clsvec = jnp.sum(labt_ref[:, :] * cls_col, axis=0, keepdims=True)
    gcls = gsum(clsvec)

    ex_w = sx2 - sx1 + 1.0
    ex_h = sy2 - sy1 + 1.0
    gt_w = gx2 - gx1 + 1.0
    gt_h = gy2 - gy1 + 1.0
    tx = (gx1 + 0.5 * gt_w - sx1 - 0.5 * ex_w) / ex_w
    ty = (gy1 + 0.5 * gt_h - sy1 - 0.5 * ex_h) / ex_h
    tw = jnp.log(gt_w / ex_w)
    th = jnp.log(gt_h / ex_h)

    rank = jax.lax.broadcasted_iota(jnp.int32, (_K, 1), 0)
    is_fg = (m_s >= 0.5) & (rank < _FG)

    cls_iota = jax.lax.broadcasted_iota(
        jnp.int32, (1, _NUM_CLASSES), 1).astype(jnp.float32)
    glab = (gcls == cls_iota).astype(jnp.float32)
    bg_row = (cls_iota == 0.0).astype(jnp.float32)
    labels_out[:, :] = jnp.where(is_fg, glab, bg_row)

    cls_i = jnp.where(is_fg, gcls.astype(jnp.int32), 0)

    bb_iota = jax.lax.broadcasted_iota(jnp.int32, (1, 4 * _NUM_CLASSES), 1)
    tsel = bb_iota % 4
    tvals = jnp.where(tsel == 0, tx,
                      jnp.where(tsel == 1, ty,
                                jnp.where(tsel == 2, tw, th)))
    bbox_out[:, :] = jnp.where((bb_iota // 4 == cls_i) & is_fg, tvals, 0.0)

    rois_out[:, :] = jnp.concatenate([sx1, sy1, sx2, sy2], axis=1)


@jax.jit
def kernel(proposals, bounding_boxes, labels):
    rois = jnp.concatenate([proposals[0], bounding_boxes[0]], axis=0)
    pad = jnp.full((_PAD - _N_ROIS, 4), -1e6, jnp.float32)
    planes = jnp.concatenate([rois, pad], axis=0).T.reshape(4, _ROWS, _LANES)
    gt = bounding_boxes[0]
    gt_t = gt.T
    lab_t = labels[0].T
    out_shape = [
        jax.ShapeDtypeStruct((_K, 4), jnp.float32),
        jax.ShapeDtypeStruct((_K, _NUM_CLASSES), jnp.float32),
        jax.ShapeDtypeStruct((_K, 4 * _NUM_CLASSES), jnp.float32),
    ]
    rois_o, labels_o, bbox_o = pl.pallas_call(
        _body,
        out_shape=out_shape,
        in_specs=[
            pl.BlockSpec(memory_space=pltpu.VMEM),
            pl.BlockSpec(memory_space=pltpu.SMEM),
            pl.BlockSpec(memory_space=pltpu.VMEM),
            pl.BlockSpec(memory_space=pltpu.VMEM),
        ],
        out_specs=[
            pl.BlockSpec(memory_space=pltpu.VMEM),
            pl.BlockSpec(memory_space=pltpu.VMEM),
            pl.BlockSpec(memory_space=pltpu.VMEM),
        ],
    )(planes, gt, gt_t, lab_t)
    return rois_o, labels_o, bbox_o
